# Initial kernel scaffold; baseline (speedup 1.0000x reference)
import functools

import jax
import jax.numpy as jnp
from jax import lax
from jax.experimental import pallas as pl
from jax.experimental.pallas import tpu as pltpu
from jax.experimental.pallas import tpu_sc as plsc

_N = 10000
_E = 320000
_L = 100000
_D = 128
_EPS = 1e-5

_NC = 2
_NS = 16
_NW = _NC * _NS

_EW = _E // _NW
_EC, _EK = 100, 100
_LW = _L // _NW
_LC, _LK = 25, 125
_RT = _N // _NS


def _sc_mesh():
    return plsc.VectorSubcoreMesh(
        core_axis_name="c", subcore_axis_name="s",
        num_cores=_NC, num_subcores=_NS)


def _zero_rows(ref, nrows, ncols16):
    def row(i, _):
        for j in range(ncols16):
            ref[i, pl.ds(j * 16, 16)] = jnp.zeros((16,), jnp.float32)
        return 0
    lax.fori_loop(0, nrows, row, 0)


def _agg_builder(with_deg):
    out_type = [jax.ShapeDtypeStruct((_NC, _N, _D), jnp.float32)]
    scratch = [
        pltpu.VMEM((_EC, _EK), jnp.int32),
        pltpu.VMEM((_EC, _EK), jnp.int32),
        pltpu.VMEM((_EK, _D), jnp.float32),
        pltpu.VMEM((_EK, _D), jnp.float32),
        pltpu.VMEM((125, _D), jnp.float32),
        pltpu.VMEM_SHARED((_N, _D), jnp.float32),
        pltpu.SemaphoreType.DMA,
        pltpu.SemaphoreType.DMA,
    ]
    if with_deg:
        out_type.append(jax.ShapeDtypeStruct((_NC, _N, 16), jnp.float32))
        scratch += [
            pltpu.VMEM((_EK, 16), jnp.float32),
            pltpu.VMEM((_RT, 16), jnp.float32),
            pltpu.VMEM_SHARED((_N, 16), jnp.float32),
        ]

    def body(x_hbm, src_hbm, dst_hbm, *rest):
        if with_deg:
            (agg_out, deg_out, sidx, didx, bufa, bufb, zbuf, acc,
             sema, semb, ones, zb16, dacc) = rest
        else:
            (agg_out, sidx, didx, bufa, bufb, zbuf, acc, sema, semb) = rest
        c = lax.axis_index("c")
        s = lax.axis_index("s")
        wid = c * _NS + s
        base = s * _RT

        _zero_rows(zbuf, 125, _D // 16)
        for t in range(_RT // 125):
            pltpu.sync_copy(zbuf, acc.at[pl.ds(base + t * 125, 125)])
        if with_deg:
            def orow(i, _):
                ones[i, :] = jnp.ones((16,), jnp.float32)
                return 0
            lax.fori_loop(0, _EK, orow, 0)
            _zero_rows(zb16, _RT, 1)
            pltpu.sync_copy(zb16, dacc.at[pl.ds(base, _RT)])
        plsc.subcore_barrier()

        pltpu.sync_copy(src_hbm.at[wid], sidx)
        pltpu.sync_copy(dst_hbm.at[wid], didx)

        pltpu.async_copy(x_hbm.at[sidx.at[0]], bufa, sema)

        def step(g, _):
            j0 = g * 2
            for b in range(2):
                j = j0 + b
                buf, sem = (bufa, sema) if b == 0 else (bufb, semb)
                nbuf, nsem = (bufb, semb) if b == 0 else (bufa, sema)
                pltpu.make_async_copy(x_hbm.at[pl.ds(0, _EK)], buf, sem).wait()

                @pl.when(j + 1 < _EC)
                def _():
                    pltpu.async_copy(x_hbm.at[sidx.at[j + 1]], nbuf, nsem)

                pltpu.sync_copy(buf, acc.at[didx.at[j]], add=True)
                if with_deg:
                    pltpu.sync_copy(ones, dacc.at[didx.at[j]], add=True)
            return 0
        lax.fori_loop(0, _EC // 2, step, 0)
        plsc.subcore_barrier()

        for t in range(_RT // 125):
            r0 = base + t * 125
            pltpu.sync_copy(acc.at[pl.ds(r0, 125)],
                            agg_out.at[c, pl.ds(r0, 125)])
        if with_deg:
            pltpu.sync_copy(dacc.at[pl.ds(base, _RT)],
                            deg_out.at[c, pl.ds(base, _RT)])

    return pl.kernel(body, out_type=out_type, mesh=_sc_mesh(),
                     scratch_types=scratch)


def _pair_gather(u, v, esr, edr):
    out_type = [jax.ShapeDtypeStruct((_L, _D), jnp.float32),
                jax.ShapeDtypeStruct((_L, _D), jnp.float32)]
    scratch = [
        pltpu.VMEM((_LC, _LK), jnp.int32),
        pltpu.VMEM((_LC, _LK), jnp.int32),
        pltpu.VMEM((_LK, _D), jnp.float32),
        pltpu.VMEM((_LK, _D), jnp.float32),
        pltpu.SemaphoreType.DMA,
        pltpu.SemaphoreType.DMA,
    ]

    def body(u_hbm, v_hbm, es_hbm, ed_hbm, su_out, sv_out,
             sidx, didx, bufu, bufv, semu, semv):
        c = lax.axis_index("c")
        s = lax.axis_index("s")
        wid = c * _NS + s
        base = wid * _LW
        pltpu.sync_copy(es_hbm.at[wid], sidx)
        pltpu.sync_copy(ed_hbm.at[wid], didx)

        def step(j, _):
            cu = pltpu.async_copy(u_hbm.at[sidx.at[j]], bufu, semu)
            cv = pltpu.async_copy(v_hbm.at[didx.at[j]], bufv, semv)
            cu.wait()
            cv.wait()
            r0 = base + j * _LK
            pltpu.sync_copy(bufu, su_out.at[pl.ds(r0, _LK)])
            pltpu.sync_copy(bufv, sv_out.at[pl.ds(r0, _LK)])
            return 0
        lax.fori_loop(0, _LC, step, 0)

    return pl.kernel(body, out_type=out_type, mesh=_sc_mesh(),
                     scratch_types=scratch)(u, v, esr, edr)


_NB = 1000


def _mean(p0, p1, d0, d1):
    deg = jnp.maximum(d0[:, 0:1] + d1[:, 0:1], 1.0)
    return (p0 + p1) / deg


def _layer1_body(p0, p1, d0, d1, x, wl, wr, bl, gs, bt, o):
    mean = _mean(p0[...], p1[...], d0[...], d1[...])
    pre = (jnp.dot(mean, wl[...], preferred_element_type=jnp.float32)
           + jnp.dot(x[...], wr[...], preferred_element_type=jnp.float32)
           + bl[...])
    o[...] = jnp.maximum(pre, 0.0) * gs[...] + bt[...]


def _layer2_body(p0, p1, d0, d1, x, wl, wr, bl, gs, bt, res, o):
    mean = _mean(p0[...], p1[...], d0[...], d1[...])
    pre = (jnp.dot(mean, wl[...], preferred_element_type=jnp.float32)
           + jnp.dot(x[...], wr[...], preferred_element_type=jnp.float32)
           + bl[...])
    o[...] = jnp.maximum(pre, 0.0) * gs[...] + bt[...] + res[...]


def _layer3_body(p0, p1, d0, d1, x, wl, wr, bl, wa, wb, u, v):
    mean = _mean(p0[...], p1[...], d0[...], d1[...])
    x3 = (jnp.dot(mean, wl[...], preferred_element_type=jnp.float32)
          + jnp.dot(x[...], wr[...], preferred_element_type=jnp.float32)
          + bl[...])
    u[...] = jnp.dot(x3, wa[...], preferred_element_type=jnp.float32)
    v[...] = jnp.dot(x3, wb[...], preferred_element_type=jnp.float32)


def _row_spec(w):
    return pl.BlockSpec((_NB, w), lambda i: (i, 0))


def _full_spec(r, w):
    return pl.BlockSpec((r, w), lambda i: (0, 0))


def _run_layer(body, nout, args):
    specs = [_row_spec(_D), _row_spec(_D), _row_spec(16), _row_spec(16),
             _row_spec(_D)]
    for a in args[5:]:
        specs.append(_row_spec(a.shape[1]) if a.shape[0] == _N
                     else _full_spec(*a.shape))
    out_shape = [jax.ShapeDtypeStruct((_N, _D), jnp.float32)] * nout
    out_specs = [_row_spec(_D)] * nout
    if nout == 1:
        out_shape, out_specs = out_shape[0], out_specs[0]
    return pl.pallas_call(
        body, grid=(_N // _NB,), in_specs=specs,
        out_specs=out_specs, out_shape=out_shape)(*args)


_BL = 2000


def _mlp_body(su, sv, b1, w2, b2, o):
    h = jnp.maximum(su[...] + sv[...] + b1[...], 0.0)
    o[...] = jnp.dot(h, w2[...], preferred_element_type=jnp.float32) + b2[...]


def _run_mlp(su, sv, b1, w2, b2):
    return pl.pallas_call(
        _mlp_body, grid=(_L // _BL,),
        in_specs=[pl.BlockSpec((_BL, _D), lambda i: (i, 0)),
                  pl.BlockSpec((_BL, _D), lambda i: (i, 0)),
                  pl.BlockSpec((1, _D), lambda i: (0, 0)),
                  pl.BlockSpec((_D, 1), lambda i: (0, 0)),
                  pl.BlockSpec((1, 1), lambda i: (0, 0))],
        out_specs=pl.BlockSpec((_BL, 1), lambda i: (i, 0)),
        out_shape=jax.ShapeDtypeStruct((_L, 1), jnp.float32))(
            su, sv, b1, w2, b2)


def kernel(x, edge_index, edge_label_index, W1l, b1l, W1r, g1, bt1,
           W2l, b2l, W2r, g2, bt2, W3l, b3l, W3r, Wp1, bp1, Wp2, bp2):
    srcr = edge_index[0].reshape(_NW, _EC, _EK)
    dstr = edge_index[1].reshape(_NW, _EC, _EK)
    esr = edge_label_index[0].reshape(_NW, _LC, _LK)
    edr = edge_label_index[1].reshape(_NW, _LC, _LK)

    inv = 1.0 / jnp.sqrt(1.0 + _EPS)
    gs1 = (g1 * inv).reshape(1, _D)
    gs2 = (g2 * inv).reshape(1, _D)
    b1l_ = b1l.reshape(1, _D)
    b2l_ = b2l.reshape(1, _D)
    b3l_ = b3l.reshape(1, _D)
    bt1_ = bt1.reshape(1, _D)
    bt2_ = bt2.reshape(1, _D)
    bp1_ = bp1.reshape(1, _D)
    bp2_ = bp2.reshape(1, 1)

    agg1, deg = _agg_builder(True)(x, srcr, dstr)
    x1 = _run_layer(_layer1_body, 1,
                    (agg1[0], agg1[1], deg[0], deg[1], x,
                     W1l, W1r, b1l_, gs1, bt1_))
    agg2, = _agg_builder(False)(x1, srcr, dstr)
    x2 = _run_layer(_layer2_body, 1,
                    (agg2[0], agg2[1], deg[0], deg[1], x1,
                     W2l, W2r, b2l_, gs2, bt2_, x1))
    agg3, = _agg_builder(False)(x2, srcr, dstr)
    u, v = _run_layer(_layer3_body, 2,
                      (agg3[0], agg3[1], deg[0], deg[1], x2,
                       W3l, W3r, b3l_, Wp1[:_D], Wp1[_D:]))
    su, sv = _pair_gather(u, v, esr, edr)
    out = _run_mlp(su, sv, bp1_, Wp2, bp2_)
    return out.reshape(-1)


# speedup vs baseline: 2.8205x; 2.8205x over previous
import jax
import jax.numpy as jnp
from jax import lax
from jax.experimental import pallas as pl
from jax.experimental.pallas import tpu as pltpu
from jax.experimental.pallas import tpu_sc as plsc

_N = 10000
_E = 320000
_L = 100000
_D = 128
_EPS = 1e-5

_NC = 2
_NS = 16
_NW = _NC * _NS

_NP = 10240
_EW = _E // _NW
_EC, _EK = 80, 128
_EWP = _EC * _EK
_LC, _LK = 32, 104
_LWP = _LC * _LK
_LP = _NW * _LWP
_RT = _NP // _NS


def _sc_mesh():
    return plsc.VectorSubcoreMesh(
        core_axis_name="c", subcore_axis_name="s",
        num_cores=_NC, num_subcores=_NS)


def _zero_rows(ref, nrows, ncols16):
    def row(i, _):
        for j in range(ncols16):
            ref[i, pl.ds(j * 16, 16)] = jnp.zeros((16,), jnp.float32)
        return 0
    lax.fori_loop(0, nrows, row, 0)


def _agg_builder():
    out_type = [jax.ShapeDtypeStruct((_NC, _NP, _D), jnp.float32)]
    scratch = [
        pltpu.VMEM((_EC, _EK), jnp.int32),
        pltpu.VMEM((_EK,), jnp.int32),
        pltpu.VMEM((_EK,), jnp.int32),
        pltpu.VMEM((_EK, _D), jnp.float32),
        pltpu.VMEM((_EK, _D), jnp.float32),
        pltpu.VMEM_SHARED((_NP, _D), jnp.float32),
        pltpu.SemaphoreType.DMA,
        pltpu.SemaphoreType.DMA,
        pltpu.SemaphoreType.DMA,
        pltpu.SemaphoreType.DMA,
    ]

    def body(x_hbm, src_hbm, dst_hbm, agg_out, sidx, db0, db1, gb0, gb1,
             acc, sg0, sg1, sd0, sd1):
        c = lax.axis_index("c")
        s = lax.axis_index("s")
        wid = c * _NS + s
        base = s * _RT
        dbuf = (db0, db1)
        gbuf = (gb0, gb1)
        gsem = (sg0, sg1)
        dsem = (sd0, sd1)

        _zero_rows(gb0, _EK, _D // 16)
        for t in range(_RT // _EK):
            pltpu.sync_copy(gb0, acc.at[pl.ds(base + t * _EK, _EK)])
        plsc.subcore_barrier()

        pltpu.sync_copy(src_hbm.at[wid], sidx)
        pltpu.async_copy(x_hbm.at[sidx.at[0]], gb0, sg0)
        pltpu.async_copy(dst_hbm.at[wid, 0], db0, sd0)

        def step(g, _):
            j0 = g * 2
            for b in range(2):
                j = j0 + b
                o = 1 - b
                pltpu.make_async_copy(
                    x_hbm.at[sidx.at[j]], gbuf[b], gsem[b]).wait()

                @pl.when(j + 1 < _EC)
                def _():
                    pltpu.async_copy(
                        x_hbm.at[sidx.at[j + 1]], gbuf[o], gsem[o])
                    pltpu.async_copy(dst_hbm.at[wid, j + 1], dbuf[o], dsem[o])

                pltpu.make_async_copy(
                    dst_hbm.at[wid, 0], dbuf[b], dsem[b]).wait()
                pltpu.sync_copy(gbuf[b], acc.at[dbuf[b]], add=True)
            return 0
        lax.fori_loop(0, _EC // 2, step, 0)
        plsc.subcore_barrier()

        for t in range(_RT // 128):
            r0 = base + t * 128
            pltpu.sync_copy(acc.at[pl.ds(r0, 128)],
                            agg_out.at[c, pl.ds(r0, 128)])

    return pl.kernel(body, out_type=out_type, mesh=_sc_mesh(),
                     scratch_types=scratch)


def _deg_builder():
    out_type = [jax.ShapeDtypeStruct((_NC, _NP, _D), jnp.float32)]
    scratch = [
        pltpu.VMEM((_EC, _EK), jnp.int32),
        pltpu.VMEM((_EK, _D), jnp.float32),
        pltpu.VMEM_SHARED((_NP, _D), jnp.float32),
    ]

    def body(dst_hbm, deg_out, didx, buf, dacc):
        c = lax.axis_index("c")
        s = lax.axis_index("s")
        wid = c * _NS + s
        base = s * _RT

        _zero_rows(buf, _EK, _D // 16)
        for t in range(_RT // _EK):
            pltpu.sync_copy(buf, dacc.at[pl.ds(base + t * _EK, _EK)])

        def orow(i, _):
            for j in range(_D // 16):
                buf[i, pl.ds(j * 16, 16)] = jnp.ones((16,), jnp.float32)
            return 0
        lax.fori_loop(0, _EK, orow, 0)
        plsc.subcore_barrier()

        pltpu.sync_copy(dst_hbm.at[wid], didx)

        def step(j, _):
            pltpu.sync_copy(buf, dacc.at[didx.at[j]], add=True)
            return 0
        lax.fori_loop(0, _EC, step, 0)
        plsc.subcore_barrier()

        for t in range(_RT // 128):
            r0 = base + t * 128
            pltpu.sync_copy(dacc.at[pl.ds(r0, 128)],
                            deg_out.at[c, pl.ds(r0, 128)])

    return pl.kernel(body, out_type=out_type, mesh=_sc_mesh(),
                     scratch_types=scratch)


def _pair_gather(u, v, esr, edr):
    out_type = [jax.ShapeDtypeStruct((_LP, _D), jnp.float32),
                jax.ShapeDtypeStruct((_LP, _D), jnp.float32)]
    scratch = [
        pltpu.VMEM((_LC, _LK), jnp.int32),
        pltpu.VMEM((_LC, _LK), jnp.int32),
        pltpu.VMEM((_LK, _D), jnp.float32),
        pltpu.VMEM((_LK, _D), jnp.float32),
        pltpu.SemaphoreType.DMA,
        pltpu.SemaphoreType.DMA,
    ]

    def body(u_hbm, v_hbm, es_hbm, ed_hbm, su_out, sv_out,
             sidx, didx, bufu, bufv, semu, semv):
        c = lax.axis_index("c")
        s = lax.axis_index("s")
        wid = c * _NS + s
        base = wid * _LWP
        pltpu.sync_copy(es_hbm.at[wid], sidx)
        pltpu.sync_copy(ed_hbm.at[wid], didx)

        def step(j, _):
            cu = pltpu.async_copy(u_hbm.at[sidx.at[j]], bufu, semu)
            cv = pltpu.async_copy(v_hbm.at[didx.at[j]], bufv, semv)
            cu.wait()
            cv.wait()
            r0 = base + j * _LK
            pltpu.sync_copy(bufu, su_out.at[pl.ds(r0, _LK)])
            pltpu.sync_copy(bufv, sv_out.at[pl.ds(r0, _LK)])
            return 0
        lax.fori_loop(0, _LC, step, 0)

    return pl.kernel(body, out_type=out_type, mesh=_sc_mesh(),
                     scratch_types=scratch)(u, v, esr, edr)


_NB = 1000


def _mean(p0, p1, d0, d1):
    deg = jnp.maximum(d0[:, 0:1] + d1[:, 0:1], 1.0)
    return (p0 + p1) / deg


def _layer1_body(p0, p1, d0, d1, x, wl, wr, bl, gs, bt, o):
    mean = _mean(p0[...], p1[...], d0[...], d1[...])
    pre = (jnp.dot(mean, wl[...], preferred_element_type=jnp.float32)
           + jnp.dot(x[...], wr[...], preferred_element_type=jnp.float32)
           + bl[...])
    o[...] = jnp.maximum(pre, 0.0) * gs[...] + bt[...]


def _layer2_body(p0, p1, d0, d1, x, wl, wr, bl, gs, bt, res, o):
    mean = _mean(p0[...], p1[...], d0[...], d1[...])
    pre = (jnp.dot(mean, wl[...], preferred_element_type=jnp.float32)
           + jnp.dot(x[...], wr[...], preferred_element_type=jnp.float32)
           + bl[...])
    o[...] = jnp.maximum(pre, 0.0) * gs[...] + bt[...] + res[...]


def _layer3_body(p0, p1, d0, d1, x, wl, wr, bl, wa, wb, u, v):
    mean = _mean(p0[...], p1[...], d0[...], d1[...])
    x3 = (jnp.dot(mean, wl[...], preferred_element_type=jnp.float32)
          + jnp.dot(x[...], wr[...], preferred_element_type=jnp.float32)
          + bl[...])
    u[...] = jnp.dot(x3, wa[...], preferred_element_type=jnp.float32)
    v[...] = jnp.dot(x3, wb[...], preferred_element_type=jnp.float32)


def _row_spec(w):
    return pl.BlockSpec((_NB, w), lambda i: (i, 0))


def _full_spec(r, w):
    return pl.BlockSpec((r, w), lambda i: (0, 0))


def _run_layer(body, nout, args):
    specs = [_row_spec(_D), _row_spec(_D), _row_spec(_D), _row_spec(_D),
             _row_spec(_D)]
    for a in args[5:]:
        specs.append(_row_spec(a.shape[1]) if a.shape[0] == _N
                     else _full_spec(*a.shape))
    out_shape = [jax.ShapeDtypeStruct((_N, _D), jnp.float32)] * nout
    out_specs = [_row_spec(_D)] * nout
    if nout == 1:
        out_shape, out_specs = out_shape[0], out_specs[0]
    return pl.pallas_call(
        body, grid=(_N // _NB,), in_specs=specs,
        out_specs=out_specs, out_shape=out_shape)(*args)


_BL = 2000


def _mlp_body(su, sv, b1, w2, b2, o):
    h = jnp.maximum(su[...] + sv[...] + b1[...], 0.0)
    o[...] = jnp.dot(h, w2[...], preferred_element_type=jnp.float32) + b2[...]


def _run_mlp(su, sv, b1, w2, b2):
    return pl.pallas_call(
        _mlp_body, grid=(_L // _BL,),
        in_specs=[pl.BlockSpec((_BL, _D), lambda i: (i, 0)),
                  pl.BlockSpec((_BL, _D), lambda i: (i, 0)),
                  pl.BlockSpec((1, _D), lambda i: (0, 0)),
                  pl.BlockSpec((_D, 1), lambda i: (0, 0)),
                  pl.BlockSpec((1, 1), lambda i: (0, 0))],
        out_specs=pl.BlockSpec((_BL, 1), lambda i: (i, 0)),
        out_shape=jax.ShapeDtypeStruct((_L, 1), jnp.float32))(
            su, sv, b1, w2, b2)


def _pad_edges(idx, fill, nw, per_w, nc, nk):
    pad = nc * nk - per_w
    r = idx.reshape(nw, per_w)
    r = jnp.concatenate(
        [r, jnp.full((nw, pad), fill, dtype=jnp.int32)], axis=1)
    return r.reshape(nw, nc, nk)


def kernel(x, edge_index, edge_label_index, W1l, b1l, W1r, g1, bt1,
           W2l, b2l, W2r, g2, bt2, W3l, b3l, W3r, Wp1, bp1, Wp2, bp2):
    srcr = _pad_edges(edge_index[0], 0, _NW, _EW, _EC, _EK)
    dstr = _pad_edges(edge_index[1], _N, _NW, _EW, _EC, _EK)
    lpad = jnp.zeros((_LP - _L,), dtype=jnp.int32)
    esr = jnp.concatenate([edge_label_index[0], lpad]).reshape(_NW, _LC, _LK)
    edr = jnp.concatenate([edge_label_index[1], lpad]).reshape(_NW, _LC, _LK)

    inv = 1.0 / jnp.sqrt(1.0 + _EPS)
    gs1 = (g1 * inv).reshape(1, _D)
    gs2 = (g2 * inv).reshape(1, _D)
    b1l_ = b1l.reshape(1, _D)
    b2l_ = b2l.reshape(1, _D)
    b3l_ = b3l.reshape(1, _D)
    bt1_ = bt1.reshape(1, _D)
    bt2_ = bt2.reshape(1, _D)
    bp1_ = bp1.reshape(1, _D)
    bp2_ = bp2.reshape(1, 1)

    deg, = _deg_builder()(dstr)
    agg1, = _agg_builder()(x, srcr, dstr)
    x1 = _run_layer(_layer1_body, 1,
                    (agg1[0], agg1[1], deg[0], deg[1], x,
                     W1l, W1r, b1l_, gs1, bt1_))
    agg2, = _agg_builder()(x1, srcr, dstr)
    x2 = _run_layer(_layer2_body, 1,
                    (agg2[0], agg2[1], deg[0], deg[1], x1,
                     W2l, W2r, b2l_, gs2, bt2_, x1))
    agg3, = _agg_builder()(x2, srcr, dstr)
    u, v = _run_layer(_layer3_body, 2,
                      (agg3[0], agg3[1], deg[0], deg[1], x2,
                       W3l, W3r, b3l_, Wp1[:_D], Wp1[_D:]))
    su, sv = _pair_gather(u, v, esr, edr)
    out = _run_mlp(su, sv, bp1_, Wp2, bp2_)
    return out.reshape(-1)


# speedup vs baseline: 2.9659x; 1.0515x over previous
import jax
import jax.numpy as jnp
from jax import lax
from jax.experimental import pallas as pl
from jax.experimental.pallas import tpu as pltpu
from jax.experimental.pallas import tpu_sc as plsc

_N = 10000
_E = 320000
_L = 100000
_D = 128
_EPS = 1e-5

_NC = 2
_NS = 16
_NW = _NC * _NS

_NP = 10240
_EW = _E // _NW
_EC, _EK = 80, 128
_EWP = _EC * _EK
_LC, _LK = 32, 104
_LWP = _LC * _LK
_LP = _NW * _LWP
_RT = _NP // _NS


def _sc_mesh():
    return plsc.VectorSubcoreMesh(
        core_axis_name="c", subcore_axis_name="s",
        num_cores=_NC, num_subcores=_NS)


def _zero_rows(ref, nrows, ncols16):
    def row(i, _):
        for j in range(ncols16):
            ref[i, pl.ds(j * 16, 16)] = jnp.zeros((16,), jnp.float32)
        return 0
    lax.fori_loop(0, nrows, row, 0)


_GD = 4
_SK = 64
_SC_CHUNKS = _EC * (_EK // _SK)


def _agg_builder():
    out_type = [jax.ShapeDtypeStruct((_NC, _NP, _D), jnp.float32)]
    scratch = [
        pltpu.VMEM((_EC, _EK), jnp.int32),
        [pltpu.VMEM((_SK,), jnp.int32) for _ in range(_GD)],
        [pltpu.VMEM((_SK, _D), jnp.float32) for _ in range(_GD)],
        pltpu.VMEM_SHARED((_NP, _D), jnp.float32),
        [pltpu.SemaphoreType.DMA for _ in range(_GD)],
        [pltpu.SemaphoreType.DMA for _ in range(_GD)],
    ]

    def body(x_hbm, src_hbm, dst_hbm, agg_out, sidx, dbuf, gbuf,
             acc, gsem, dsem):
        c = lax.axis_index("c")
        s = lax.axis_index("s")
        wid = c * _NS + s
        base = s * _RT

        def sidx_sl(j):
            return sidx.at[lax.div(j, 2), pl.ds(lax.rem(j, 2) * _SK, _SK)]

        def didx_sl(j):
            return dst_hbm.at[wid, lax.div(j, 2),
                              pl.ds(lax.rem(j, 2) * _SK, _SK)]

        _zero_rows(gbuf[0], _SK, _D // 16)
        for t in range(_RT // _SK):
            pltpu.sync_copy(gbuf[0], acc.at[pl.ds(base + t * _SK, _SK)])
        plsc.subcore_barrier()

        pltpu.sync_copy(src_hbm.at[wid], sidx)
        for j in range(_GD - 1):
            pltpu.async_copy(x_hbm.at[sidx_sl(j)], gbuf[j], gsem[j])
            pltpu.async_copy(didx_sl(j), dbuf[j], dsem[j])

        def step(g, _):
            j0 = g * _GD
            for b in range(_GD):
                j = j0 + b
                pltpu.make_async_copy(
                    x_hbm.at[sidx_sl(j)], gbuf[b], gsem[b]).wait()

                @pl.when(j + _GD - 1 < _SC_CHUNKS)
                def _():
                    nb = (b + _GD - 1) % _GD
                    pltpu.async_copy(
                        x_hbm.at[sidx_sl(j + _GD - 1)], gbuf[nb], gsem[nb])
                    pltpu.async_copy(didx_sl(j + _GD - 1), dbuf[nb], dsem[nb])

                pltpu.make_async_copy(
                    didx_sl(j), dbuf[b], dsem[b]).wait()
                pltpu.sync_copy(gbuf[b], acc.at[dbuf[b]], add=True)
            return 0
        lax.fori_loop(0, _SC_CHUNKS // _GD, step, 0)
        plsc.subcore_barrier()

        for t in range(_RT // 128):
            r0 = base + t * 128
            pltpu.sync_copy(acc.at[pl.ds(r0, 128)],
                            agg_out.at[c, pl.ds(r0, 128)])

    return pl.kernel(body, out_type=out_type, mesh=_sc_mesh(),
                     scratch_types=scratch)


def _deg_builder():
    out_type = [jax.ShapeDtypeStruct((_NC, _NP, _D), jnp.float32)]
    scratch = [
        pltpu.VMEM((_EC, _EK), jnp.int32),
        pltpu.VMEM((_EK, _D), jnp.float32),
        pltpu.VMEM_SHARED((_NP, _D), jnp.float32),
    ]

    def body(dst_hbm, deg_out, didx, buf, dacc):
        c = lax.axis_index("c")
        s = lax.axis_index("s")
        wid = c * _NS + s
        base = s * _RT

        _zero_rows(buf, _EK, _D // 16)
        for t in range(_RT // _EK):
            pltpu.sync_copy(buf, dacc.at[pl.ds(base + t * _EK, _EK)])

        def orow(i, _):
            for j in range(_D // 16):
                buf[i, pl.ds(j * 16, 16)] = jnp.ones((16,), jnp.float32)
            return 0
        lax.fori_loop(0, _EK, orow, 0)
        plsc.subcore_barrier()

        pltpu.sync_copy(dst_hbm.at[wid], didx)

        def step(j, _):
            pltpu.sync_copy(buf, dacc.at[didx.at[j]], add=True)
            return 0
        lax.fori_loop(0, _EC, step, 0)
        plsc.subcore_barrier()

        for t in range(_RT // 128):
            r0 = base + t * 128
            pltpu.sync_copy(dacc.at[pl.ds(r0, 128)],
                            deg_out.at[c, pl.ds(r0, 128)])

    return pl.kernel(body, out_type=out_type, mesh=_sc_mesh(),
                     scratch_types=scratch)


def _pair_gather(u, v, esr, edr):
    out_type = [jax.ShapeDtypeStruct((_LP, _D), jnp.float32),
                jax.ShapeDtypeStruct((_LP, _D), jnp.float32)]
    scratch = [
        pltpu.VMEM((_LC, _LK), jnp.int32),
        pltpu.VMEM((_LC, _LK), jnp.int32),
        pltpu.VMEM((_LK, _D), jnp.float32),
        pltpu.VMEM((_LK, _D), jnp.float32),
        pltpu.SemaphoreType.DMA,
        pltpu.SemaphoreType.DMA,
    ]

    def body(u_hbm, v_hbm, es_hbm, ed_hbm, su_out, sv_out,
             sidx, didx, bufu, bufv, semu, semv):
        c = lax.axis_index("c")
        s = lax.axis_index("s")
        wid = c * _NS + s
        base = wid * _LWP
        pltpu.sync_copy(es_hbm.at[wid], sidx)
        pltpu.sync_copy(ed_hbm.at[wid], didx)

        def step(j, _):
            cu = pltpu.async_copy(u_hbm.at[sidx.at[j]], bufu, semu)
            cv = pltpu.async_copy(v_hbm.at[didx.at[j]], bufv, semv)
            cu.wait()
            cv.wait()
            r0 = base + j * _LK
            pltpu.sync_copy(bufu, su_out.at[pl.ds(r0, _LK)])
            pltpu.sync_copy(bufv, sv_out.at[pl.ds(r0, _LK)])
            return 0
        lax.fori_loop(0, _LC, step, 0)

    return pl.kernel(body, out_type=out_type, mesh=_sc_mesh(),
                     scratch_types=scratch)(u, v, esr, edr)


_NB = 1000


def _mean(p0, p1, d0, d1):
    deg = jnp.maximum(d0[:, 0:1] + d1[:, 0:1], 1.0)
    return (p0 + p1) / deg


def _layer1_body(p0, p1, d0, d1, x, wl, wr, bl, gs, bt, o):
    mean = _mean(p0[...], p1[...], d0[...], d1[...])
    pre = (jnp.dot(mean, wl[...], preferred_element_type=jnp.float32)
           + jnp.dot(x[...], wr[...], preferred_element_type=jnp.float32)
           + bl[...])
    o[...] = jnp.maximum(pre, 0.0) * gs[...] + bt[...]


def _layer2_body(p0, p1, d0, d1, x, wl, wr, bl, gs, bt, res, o):
    mean = _mean(p0[...], p1[...], d0[...], d1[...])
    pre = (jnp.dot(mean, wl[...], preferred_element_type=jnp.float32)
           + jnp.dot(x[...], wr[...], preferred_element_type=jnp.float32)
           + bl[...])
    o[...] = jnp.maximum(pre, 0.0) * gs[...] + bt[...] + res[...]


def _layer3_body(p0, p1, d0, d1, x, wl, wr, bl, wa, wb, u, v):
    mean = _mean(p0[...], p1[...], d0[...], d1[...])
    x3 = (jnp.dot(mean, wl[...], preferred_element_type=jnp.float32)
          + jnp.dot(x[...], wr[...], preferred_element_type=jnp.float32)
          + bl[...])
    u[...] = jnp.dot(x3, wa[...], preferred_element_type=jnp.float32)
    v[...] = jnp.dot(x3, wb[...], preferred_element_type=jnp.float32)


def _row_spec(w):
    return pl.BlockSpec((_NB, w), lambda i: (i, 0))


def _full_spec(r, w):
    return pl.BlockSpec((r, w), lambda i: (0, 0))


def _run_layer(body, nout, args):
    specs = [_row_spec(_D), _row_spec(_D), _row_spec(_D), _row_spec(_D),
             _row_spec(_D)]
    for a in args[5:]:
        specs.append(_row_spec(a.shape[1]) if a.shape[0] == _N
                     else _full_spec(*a.shape))
    out_shape = [jax.ShapeDtypeStruct((_N, _D), jnp.float32)] * nout
    out_specs = [_row_spec(_D)] * nout
    if nout == 1:
        out_shape, out_specs = out_shape[0], out_specs[0]
    return pl.pallas_call(
        body, grid=(_N // _NB,), in_specs=specs,
        out_specs=out_specs, out_shape=out_shape)(*args)


_BL = 2000


def _mlp_body(su, sv, b1, w2, b2, o):
    h = jnp.maximum(su[...] + sv[...] + b1[...], 0.0)
    o[...] = jnp.dot(h, w2[...], preferred_element_type=jnp.float32) + b2[...]


def _run_mlp(su, sv, b1, w2, b2):
    return pl.pallas_call(
        _mlp_body, grid=(_L // _BL,),
        in_specs=[pl.BlockSpec((_BL, _D), lambda i: (i, 0)),
                  pl.BlockSpec((_BL, _D), lambda i: (i, 0)),
                  pl.BlockSpec((1, _D), lambda i: (0, 0)),
                  pl.BlockSpec((_D, 1), lambda i: (0, 0)),
                  pl.BlockSpec((1, 1), lambda i: (0, 0))],
        out_specs=pl.BlockSpec((_BL, 1), lambda i: (i, 0)),
        out_shape=jax.ShapeDtypeStruct((_L, 1), jnp.float32))(
            su, sv, b1, w2, b2)


def _pad_edges(idx, fill, nw, per_w, nc, nk):
    pad = nc * nk - per_w
    r = idx.reshape(nw, per_w)
    r = jnp.concatenate(
        [r, jnp.full((nw, pad), fill, dtype=jnp.int32)], axis=1)
    return r.reshape(nw, nc, nk)


def kernel(x, edge_index, edge_label_index, W1l, b1l, W1r, g1, bt1,
           W2l, b2l, W2r, g2, bt2, W3l, b3l, W3r, Wp1, bp1, Wp2, bp2):
    srcr = _pad_edges(edge_index[0], 0, _NW, _EW, _EC, _EK)
    dstr = _pad_edges(edge_index[1], _N, _NW, _EW, _EC, _EK)
    lpad = jnp.zeros((_LP - _L,), dtype=jnp.int32)
    esr = jnp.concatenate([edge_label_index[0], lpad]).reshape(_NW, _LC, _LK)
    edr = jnp.concatenate([edge_label_index[1], lpad]).reshape(_NW, _LC, _LK)

    inv = 1.0 / jnp.sqrt(1.0 + _EPS)
    gs1 = (g1 * inv).reshape(1, _D)
    gs2 = (g2 * inv).reshape(1, _D)
    b1l_ = b1l.reshape(1, _D)
    b2l_ = b2l.reshape(1, _D)
    b3l_ = b3l.reshape(1, _D)
    bt1_ = bt1.reshape(1, _D)
    bt2_ = bt2.reshape(1, _D)
    bp1_ = bp1.reshape(1, _D)
    bp2_ = bp2.reshape(1, 1)

    deg, = _deg_builder()(dstr)
    agg1, = _agg_builder()(x, srcr, dstr)
    x1 = _run_layer(_layer1_body, 1,
                    (agg1[0], agg1[1], deg[0], deg[1], x,
                     W1l, W1r, b1l_, gs1, bt1_))
    agg2, = _agg_builder()(x1, srcr, dstr)
    x2 = _run_layer(_layer2_body, 1,
                    (agg2[0], agg2[1], deg[0], deg[1], x1,
                     W2l, W2r, b2l_, gs2, bt2_, x1))
    agg3, = _agg_builder()(x2, srcr, dstr)
    u, v = _run_layer(_layer3_body, 2,
                      (agg3[0], agg3[1], deg[0], deg[1], x2,
                       W3l, W3r, b3l_, Wp1[:_D], Wp1[_D:]))
    su, sv = _pair_gather(u, v, esr, edr)
    out = _run_mlp(su, sv, bp1_, Wp2, bp2_)
    return out.reshape(-1)


# speedup vs baseline: 3.0021x; 1.0122x over previous
import jax
import jax.numpy as jnp
from jax import lax
from jax.experimental import pallas as pl
from jax.experimental.pallas import tpu as pltpu
from jax.experimental.pallas import tpu_sc as plsc

_N = 10000
_E = 320000
_L = 100000
_D = 128
_EPS = 1e-5

_NC = 2
_NS = 16
_NW = _NC * _NS

_NP = 10240
_EW = _E // _NW
_EC, _EK = 80, 128
_EWP = _EC * _EK
_LC, _LK = 32, 104
_LWP = _LC * _LK
_LP = _NW * _LWP
_RT = _NP // _NS


def _sc_mesh():
    return plsc.VectorSubcoreMesh(
        core_axis_name="c", subcore_axis_name="s",
        num_cores=_NC, num_subcores=_NS)


def _zero_rows(ref, nrows, ncols16):
    def row(i, _):
        for j in range(ncols16):
            ref[i, pl.ds(j * 16, 16)] = jnp.zeros((16,), jnp.float32)
        return 0
    lax.fori_loop(0, nrows, row, 0)


_GD = 4
_SK = 64
_SC_CHUNKS = _EC * (_EK // _SK)


def _agg_builder():
    out_type = [jax.ShapeDtypeStruct((_NC, _NP, _D), jnp.float32)]
    scratch = [
        pltpu.VMEM((_EC, _EK), jnp.int32),
        [pltpu.VMEM((_SK,), jnp.int32) for _ in range(_GD)],
        [pltpu.VMEM((_SK, _D), jnp.float32) for _ in range(_GD)],
        pltpu.VMEM_SHARED((_NP, _D), jnp.float32),
        [pltpu.SemaphoreType.DMA for _ in range(_GD)],
        [pltpu.SemaphoreType.DMA for _ in range(_GD)],
    ]

    def body(x_hbm, src_hbm, dst_hbm, agg_out, sidx, dbuf, gbuf,
             acc, gsem, dsem):
        c = lax.axis_index("c")
        s = lax.axis_index("s")
        wid = c * _NS + s
        base = s * _RT

        def sidx_sl(j):
            return sidx.at[lax.div(j, 2), pl.ds(lax.rem(j, 2) * _SK, _SK)]

        def didx_sl(j):
            return dst_hbm.at[wid, lax.div(j, 2),
                              pl.ds(lax.rem(j, 2) * _SK, _SK)]

        _zero_rows(gbuf[0], _SK, _D // 16)
        for t in range(_RT // _SK):
            pltpu.sync_copy(gbuf[0], acc.at[pl.ds(base + t * _SK, _SK)])
        plsc.subcore_barrier()

        pltpu.sync_copy(src_hbm.at[wid], sidx)
        for j in range(_GD - 1):
            pltpu.async_copy(x_hbm.at[sidx_sl(j)], gbuf[j], gsem[j])
            pltpu.async_copy(didx_sl(j), dbuf[j], dsem[j])

        def step(g, _):
            j0 = g * _GD
            for b in range(_GD):
                j = j0 + b
                pltpu.make_async_copy(
                    x_hbm.at[sidx_sl(j)], gbuf[b], gsem[b]).wait()

                @pl.when(j + _GD - 1 < _SC_CHUNKS)
                def _():
                    nb = (b + _GD - 1) % _GD
                    pltpu.async_copy(
                        x_hbm.at[sidx_sl(j + _GD - 1)], gbuf[nb], gsem[nb])
                    pltpu.async_copy(didx_sl(j + _GD - 1), dbuf[nb], dsem[nb])

                pltpu.make_async_copy(
                    didx_sl(j), dbuf[b], dsem[b]).wait()
                pltpu.sync_copy(gbuf[b], acc.at[dbuf[b]], add=True)
            return 0
        lax.fori_loop(0, _SC_CHUNKS // _GD, step, 0)
        plsc.subcore_barrier()

        for t in range(_RT // 128):
            r0 = base + t * 128
            pltpu.sync_copy(acc.at[pl.ds(r0, 128)],
                            agg_out.at[c, pl.ds(r0, 128)])

    return pl.kernel(body, out_type=out_type, mesh=_sc_mesh(),
                     scratch_types=scratch)


def _deg_builder():
    out_type = [jax.ShapeDtypeStruct((_NC, _NP, _D), jnp.float32)]
    scratch = [
        pltpu.VMEM((_EC, _EK), jnp.int32),
        pltpu.VMEM((_EK, _D), jnp.float32),
        pltpu.VMEM_SHARED((_NP, _D), jnp.float32),
    ]

    def body(dst_hbm, deg_out, didx, buf, dacc):
        c = lax.axis_index("c")
        s = lax.axis_index("s")
        wid = c * _NS + s
        base = s * _RT

        _zero_rows(buf, _EK, _D // 16)
        for t in range(_RT // _EK):
            pltpu.sync_copy(buf, dacc.at[pl.ds(base + t * _EK, _EK)])

        def orow(i, _):
            for j in range(_D // 16):
                buf[i, pl.ds(j * 16, 16)] = jnp.ones((16,), jnp.float32)
            return 0
        lax.fori_loop(0, _EK, orow, 0)
        plsc.subcore_barrier()

        pltpu.sync_copy(dst_hbm.at[wid], didx)

        def step(j, _):
            pltpu.sync_copy(buf, dacc.at[didx.at[j]], add=True)
            return 0
        lax.fori_loop(0, _EC, step, 0)
        plsc.subcore_barrier()

        for t in range(_RT // 128):
            r0 = base + t * 128
            pltpu.sync_copy(dacc.at[pl.ds(r0, 128)],
                            deg_out.at[c, pl.ds(r0, 128)])

    return pl.kernel(body, out_type=out_type, mesh=_sc_mesh(),
                     scratch_types=scratch)


def _pair_gather(u, v, esr, edr):
    out_type = [jax.ShapeDtypeStruct((_LP, _D), jnp.float32)]
    scratch = [
        pltpu.VMEM((_LC, _LK), jnp.int32),
        pltpu.VMEM((_LC, _LK), jnp.int32),
        [pltpu.VMEM((_LK, _D), jnp.float32) for _ in range(2)],
        [pltpu.VMEM((_LK, _D), jnp.float32) for _ in range(2)],
        [pltpu.SemaphoreType.DMA for _ in range(2)],
        [pltpu.SemaphoreType.DMA for _ in range(2)],
    ]

    def body(u_hbm, v_hbm, es_hbm, ed_hbm, s_out,
             sidx, didx, bufu, bufv, semu, semv):
        c = lax.axis_index("c")
        s = lax.axis_index("s")
        wid = c * _NS + s
        base = wid * _LWP
        pltpu.sync_copy(es_hbm.at[wid], sidx)
        pltpu.sync_copy(ed_hbm.at[wid], didx)

        pltpu.async_copy(u_hbm.at[sidx.at[0]], bufu[0], semu[0])
        pltpu.async_copy(v_hbm.at[didx.at[0]], bufv[0], semv[0])

        def step(g, _):
            j0 = g * 2
            for b in range(2):
                j = j0 + b
                o = 1 - b
                pltpu.make_async_copy(
                    u_hbm.at[sidx.at[j]], bufu[b], semu[b]).wait()
                pltpu.make_async_copy(
                    v_hbm.at[didx.at[j]], bufv[b], semv[b]).wait()

                @pl.when(j + 1 < _LC)
                def _():
                    pltpu.async_copy(
                        u_hbm.at[sidx.at[j + 1]], bufu[o], semu[o])
                    pltpu.async_copy(
                        v_hbm.at[didx.at[j + 1]], bufv[o], semv[o])

                def srow(i, _):
                    for t in range(_D // 16):
                        sl = pl.ds(t * 16, 16)
                        bufu[b][i, sl] = bufu[b][i, sl] + bufv[b][i, sl]
                    return 0
                lax.fori_loop(0, _LK, srow, 0)
                pltpu.sync_copy(bufu[b], s_out.at[pl.ds(base + j * _LK, _LK)])
            return 0
        lax.fori_loop(0, _LC // 2, step, 0)

    return pl.kernel(body, out_type=out_type, mesh=_sc_mesh(),
                     scratch_types=scratch)(u, v, esr, edr)


_NB = 1000


def _mean(p0, p1, d0, d1):
    deg = jnp.maximum(d0[:, 0:1] + d1[:, 0:1], 1.0)
    return (p0 + p1) / deg


def _layer1_body(p0, p1, d0, d1, x, wl, wr, bl, gs, bt, o):
    mean = _mean(p0[...], p1[...], d0[...], d1[...])
    pre = (jnp.dot(mean, wl[...], preferred_element_type=jnp.float32)
           + jnp.dot(x[...], wr[...], preferred_element_type=jnp.float32)
           + bl[...])
    o[...] = jnp.maximum(pre, 0.0) * gs[...] + bt[...]


def _layer2_body(p0, p1, d0, d1, x, wl, wr, bl, gs, bt, res, o):
    mean = _mean(p0[...], p1[...], d0[...], d1[...])
    pre = (jnp.dot(mean, wl[...], preferred_element_type=jnp.float32)
           + jnp.dot(x[...], wr[...], preferred_element_type=jnp.float32)
           + bl[...])
    o[...] = jnp.maximum(pre, 0.0) * gs[...] + bt[...] + res[...]


def _layer3_body(p0, p1, d0, d1, x, wl, wr, bl, wa, wb, u, v):
    mean = _mean(p0[...], p1[...], d0[...], d1[...])
    x3 = (jnp.dot(mean, wl[...], preferred_element_type=jnp.float32)
          + jnp.dot(x[...], wr[...], preferred_element_type=jnp.float32)
          + bl[...])
    u[...] = jnp.dot(x3, wa[...], preferred_element_type=jnp.float32)
    v[...] = jnp.dot(x3, wb[...], preferred_element_type=jnp.float32)


def _row_spec(w):
    return pl.BlockSpec((_NB, w), lambda i: (i, 0))


def _full_spec(r, w):
    return pl.BlockSpec((r, w), lambda i: (0, 0))


def _run_layer(body, nout, args):
    specs = [_row_spec(_D), _row_spec(_D), _row_spec(_D), _row_spec(_D),
             _row_spec(_D)]
    for a in args[5:]:
        specs.append(_row_spec(a.shape[1]) if a.shape[0] == _N
                     else _full_spec(*a.shape))
    out_shape = [jax.ShapeDtypeStruct((_N, _D), jnp.float32)] * nout
    out_specs = [_row_spec(_D)] * nout
    if nout == 1:
        out_shape, out_specs = out_shape[0], out_specs[0]
    return pl.pallas_call(
        body, grid=(_N // _NB,), in_specs=specs,
        out_specs=out_specs, out_shape=out_shape)(*args)


_BL = 2000


def _mlp_body(sm, b1, w2, b2, o):
    h = jnp.maximum(sm[...] + b1[...], 0.0)
    o[...] = jnp.dot(h, w2[...], preferred_element_type=jnp.float32) + b2[...]


def _run_mlp(sm, b1, w2, b2):
    return pl.pallas_call(
        _mlp_body, grid=(_L // _BL,),
        in_specs=[pl.BlockSpec((_BL, _D), lambda i: (i, 0)),
                  pl.BlockSpec((1, _D), lambda i: (0, 0)),
                  pl.BlockSpec((_D, 1), lambda i: (0, 0)),
                  pl.BlockSpec((1, 1), lambda i: (0, 0))],
        out_specs=pl.BlockSpec((_BL, 1), lambda i: (i, 0)),
        out_shape=jax.ShapeDtypeStruct((_L, 1), jnp.float32))(
            sm, b1, w2, b2)


def _pad_edges(idx, fill, nw, per_w, nc, nk):
    pad = nc * nk - per_w
    r = idx.reshape(nw, per_w)
    r = jnp.concatenate(
        [r, jnp.full((nw, pad), fill, dtype=jnp.int32)], axis=1)
    return r.reshape(nw, nc, nk)


def kernel(x, edge_index, edge_label_index, W1l, b1l, W1r, g1, bt1,
           W2l, b2l, W2r, g2, bt2, W3l, b3l, W3r, Wp1, bp1, Wp2, bp2):
    srcr = _pad_edges(edge_index[0], 0, _NW, _EW, _EC, _EK)
    dstr = _pad_edges(edge_index[1], _N, _NW, _EW, _EC, _EK)
    lpad = jnp.zeros((_LP - _L,), dtype=jnp.int32)
    esr = jnp.concatenate([edge_label_index[0], lpad]).reshape(_NW, _LC, _LK)
    edr = jnp.concatenate([edge_label_index[1], lpad]).reshape(_NW, _LC, _LK)

    inv = 1.0 / jnp.sqrt(1.0 + _EPS)
    gs1 = (g1 * inv).reshape(1, _D)
    gs2 = (g2 * inv).reshape(1, _D)
    b1l_ = b1l.reshape(1, _D)
    b2l_ = b2l.reshape(1, _D)
    b3l_ = b3l.reshape(1, _D)
    bt1_ = bt1.reshape(1, _D)
    bt2_ = bt2.reshape(1, _D)
    bp1_ = bp1.reshape(1, _D)
    bp2_ = bp2.reshape(1, 1)

    deg, = _deg_builder()(dstr)
    agg1, = _agg_builder()(x, srcr, dstr)
    x1 = _run_layer(_layer1_body, 1,
                    (agg1[0], agg1[1], deg[0], deg[1], x,
                     W1l, W1r, b1l_, gs1, bt1_))
    agg2, = _agg_builder()(x1, srcr, dstr)
    x2 = _run_layer(_layer2_body, 1,
                    (agg2[0], agg2[1], deg[0], deg[1], x1,
                     W2l, W2r, b2l_, gs2, bt2_, x1))
    agg3, = _agg_builder()(x2, srcr, dstr)
    u, v = _run_layer(_layer3_body, 2,
                      (agg3[0], agg3[1], deg[0], deg[1], x2,
                       W3l, W3r, b3l_, Wp1[:_D], Wp1[_D:]))
    sm, = _pair_gather(u, v, esr, edr)
    out = _run_mlp(sm, bp1_, Wp2, bp2_)
    return out.reshape(-1)
